# baseline (device time: 107015 ns/iter reference)
import jax
import jax.numpy as jnp
from jax import lax
from jax.experimental import pallas as pl
from jax.experimental.pallas import tpu as pltpu

N_DEV = 4
HTA = 256
HTB = 512
COMM = True
AG_ON = True
BF16 = jnp.bfloat16


def kernel(x, Wg, Wu, Wd):
    m, k = x.shape
    hs = Wg.shape[1]
    n = Wd.shape[1]
    gr = m // (2 * N_DEV)
    half = m // 2
    nta = hs // HTA
    ntb = hs // HTB

    def body(x_ref, wg_hbm, wu_hbm, wd_hbm, out_ref,
             wgu_buf, wd_buf, dma_sems, h_buf, xb_buf,
             comm_a, comm_b, rs_sb_a, rs_sb_b,
             ag0_a, agb_a, ag0_b, agb_b,
             rs_send, rs_recv, ag_send, ag_recv):
        d = lax.axis_index("i")
        left = lax.rem(d + N_DEV - 1, N_DEV)
        right = lax.rem(d + 1, N_DEV)

        barrier_sem = pltpu.get_barrier_semaphore()
        for nbr in (left, right):
            pl.semaphore_signal(
                barrier_sem, inc=1,
                device_id=(nbr,), device_id_type=pl.DeviceIdType.MESH,
            )

        def a_rows(j):
            c = lax.rem(d + N_DEV - j, N_DEV)
            return pl.ds(c * gr, gr)

        def b_rows(j):
            c = lax.rem(d + j, N_DEV)
            return pl.ds(half + c * gr, gr)

        def rs_bf(dirn, j):
            sb = rs_sb_a if dirn == 0 else rs_sb_b
            comm = comm_a if dirn == 0 else comm_b
            return pltpu.make_async_remote_copy(
                src_ref=sb,
                dst_ref=comm.at[j],
                send_sem=rs_send.at[dirn, j],
                recv_sem=rs_recv.at[dirn, j],
                device_id=(right if dirn == 0 else left,),
                device_id_type=pl.DeviceIdType.MESH,
            )

        def rs_boundary(j):
            if j == 0:
                pl.semaphore_wait(barrier_sem, 2)
            else:
                for dirn in (0, 1):
                    r = rs_bf(dirn, j - 1)
                    r.wait_send()
                    r.wait_recv()
                    rows = a_rows(j) if dirn == 0 else b_rows(j)
                    comm = comm_a if dirn == 0 else comm_b
                    out_ref[rows, :] = (
                        out_ref[rows, :] + comm[j - 1].astype(jnp.float32))
            if j < N_DEV - 1:
                rs_sb_a[:, :] = out_ref[a_rows(j), :].astype(BF16)
                rs_bf(0, j).start()
                rs_sb_b[:, :] = out_ref[b_rows(j), :].astype(BF16)
                rs_bf(1, j).start()

        def p0_copies(t, slot):
            cols = pl.ds(t * HTA, HTA)
            return [
                pltpu.make_async_copy(
                    wg_hbm.at[:, cols], wgu_buf.at[slot, 0], dma_sems.at[slot, 0]),
                pltpu.make_async_copy(
                    wu_hbm.at[:, cols], wgu_buf.at[slot, 1], dma_sems.at[slot, 1]),
                pltpu.make_async_copy(
                    wd_hbm.at[pl.ds(t * HTA, HTA), :],
                    wd_buf.at[slot, pl.ds(0, HTA)], dma_sems.at[slot, 2]),
            ]

        for c in p0_copies(0, 0):
            c.start()
        xb_buf[:, :] = x_ref[:, :].astype(BF16)
        for t in range(nta):
            slot = t % 2
            if t + 1 < nta:
                for c in p0_copies(t + 1, (t + 1) % 2):
                    c.start()
            for c in p0_copies(t, slot):
                c.wait()
            cols = pl.ds(t * HTA, HTA)
            for r2 in range(2):
                rows2 = pl.ds(r2 * half, half)
                gate = jnp.dot(xb_buf[rows2, :], wgu_buf[slot, 0].astype(BF16),
                               preferred_element_type=jnp.float32)
                up = jnp.dot(xb_buf[rows2, :], wgu_buf[slot, 1].astype(BF16),
                             preferred_element_type=jnp.float32)
                h_buf[rows2, cols] = (
                    gate * (up * jax.nn.sigmoid(up))).astype(BF16)
            wdt = wd_buf[slot, pl.ds(0, HTA)].astype(BF16)
            for rows in (a_rows(0), b_rows(0)):
                part = jnp.dot(h_buf[rows, cols], wdt,
                               preferred_element_type=jnp.float32)
                if t == 0:
                    out_ref[rows, :] = part
                else:
                    out_ref[rows, :] = out_ref[rows, :] + part

        if COMM:
            rs_boundary(0)

        def wd_copy(t6, slot):
            return pltpu.make_async_copy(
                wd_hbm.at[pl.ds(t6 * HTB, HTB), :], wd_buf.at[slot],
                dma_sems.at[slot, 2])

        wd_copy(0, 0).start()
        for j in range(1, N_DEV):
            for t6 in range(ntb):
                flatb = (j - 1) * ntb + t6
                slot = flatb % 2
                if flatb + 1 < (N_DEV - 1) * ntb:
                    wd_copy((t6 + 1) % ntb, (flatb + 1) % 2).start()
                wd_copy(t6, slot).wait()
                wdt = wd_buf[slot].astype(BF16)
                cols = pl.ds(t6 * HTB, HTB)
                for rows in (a_rows(j), b_rows(j)):
                    part = jnp.dot(h_buf[rows, cols], wdt,
                                   preferred_element_type=jnp.float32)
                    if t6 == 0:
                        out_ref[rows, :] = part
                    else:
                        out_ref[rows, :] = out_ref[rows, :] + part
            if COMM:
                rs_boundary(j)

        def ag_rdma(dirn, s):
            ag0 = ag0_a if dirn == 0 else ag0_b
            agb = agb_a if dirn == 0 else agb_b
            src = ag0 if s == 0 else agb.at[s - 1]
            return pltpu.make_async_remote_copy(
                src_ref=src,
                dst_ref=agb.at[s],
                send_sem=ag_send.at[dirn, s],
                recv_sem=ag_recv.at[dirn, s],
                device_id=(right if dirn == 0 else left,),
                device_id_type=pl.DeviceIdType.MESH,
            )

        def ag_recv_rows(dirn, s):
            if dirn == 0:
                c = lax.rem(d + N_DEV - s, N_DEV)
                return pl.ds(c * gr, gr)
            c = lax.rem(d + s, N_DEV)
            return pl.ds(half + c * gr, gr)

        if COMM and AG_ON:
            own_a = lax.rem(d + 1, N_DEV)
            ag0_a[:, :] = out_ref[pl.ds(own_a * gr, gr), :].astype(BF16)
            own_b = lax.rem(d + N_DEV - 1, N_DEV)
            ag0_b[:, :] = out_ref[pl.ds(half + own_b * gr, gr), :].astype(BF16)
            ag_rdma(0, 0).start()
            ag_rdma(1, 0).start()
            for s in range(N_DEV - 1):
                ag_rdma(0, s).wait_recv()
                ag_rdma(1, s).wait_recv()
                if s + 1 < N_DEV - 1:
                    ag_rdma(0, s + 1).start()
                    ag_rdma(1, s + 1).start()
                out_ref[ag_recv_rows(0, s), :] = agb_a[s].astype(jnp.float32)
                out_ref[ag_recv_rows(1, s), :] = agb_b[s].astype(jnp.float32)
            for s in range(N_DEV - 1):
                ag_rdma(0, s).wait_send()
                ag_rdma(1, s).wait_send()

    return pl.pallas_call(
        body,
        out_shape=jax.ShapeDtypeStruct((m, n), jnp.float32),
        in_specs=[
            pl.BlockSpec(memory_space=pltpu.VMEM),
            pl.BlockSpec(memory_space=pltpu.MemorySpace.HBM),
            pl.BlockSpec(memory_space=pltpu.MemorySpace.HBM),
            pl.BlockSpec(memory_space=pltpu.MemorySpace.HBM),
        ],
        out_specs=pl.BlockSpec(memory_space=pltpu.VMEM),
        scratch_shapes=[
            pltpu.VMEM((2, 2, k, HTA), jnp.float32),
            pltpu.VMEM((2, HTB, n), jnp.float32),
            pltpu.SemaphoreType.DMA((2, 3)),
            pltpu.VMEM((m, hs), BF16),
            pltpu.VMEM((m, k), BF16),
            pltpu.VMEM((N_DEV - 1, gr, n), BF16),
            pltpu.VMEM((N_DEV - 1, gr, n), BF16),
            pltpu.VMEM((gr, n), BF16),
            pltpu.VMEM((gr, n), BF16),
            pltpu.VMEM((gr, n), BF16),
            pltpu.VMEM((N_DEV - 1, gr, n), BF16),
            pltpu.VMEM((gr, n), BF16),
            pltpu.VMEM((N_DEV - 1, gr, n), BF16),
            pltpu.SemaphoreType.DMA((2, N_DEV - 1)),
            pltpu.SemaphoreType.DMA((2, N_DEV - 1)),
            pltpu.SemaphoreType.DMA((2, N_DEV - 1)),
            pltpu.SemaphoreType.DMA((2, N_DEV - 1)),
        ],
        compiler_params=pltpu.CompilerParams(
            collective_id=0,
            vmem_limit_bytes=60 * 1024 * 1024,
        ),
    )(x, Wg, Wu, Wd)
